# initial kernel scaffold (unmeasured)
import functools

import jax
import jax.numpy as jnp
from jax import lax
from jax.experimental import pallas as pl
from jax.experimental.pallas import tpu as pltpu

N_DEV = 16


def kernel(x, w_mat):
    m_glob, k_per = x.shape
    _, n = w_mat.shape
    m_per = m_glob // N_DEV

    def body(x_ref, w_ref, out_ref, send_buf, recv_buf, send_sems, recv_sems):
        my = lax.axis_index("i")
        left = lax.rem(my + N_DEV - 1, N_DEV)
        right = lax.rem(my + 1, N_DEV)

        barrier_sem = pltpu.get_barrier_semaphore()
        for nbr in (left, right):
            pl.semaphore_signal(
                barrier_sem, inc=1,
                device_id=(nbr,), device_id_type=pl.DeviceIdType.MESH,
            )
        pl.semaphore_wait(barrier_sem, 2)

        for s in range(N_DEV):
            c = lax.rem(my + 2 * N_DEV - 1 - s, N_DEV)
            part = jnp.dot(
                x_ref[pl.ds(c * m_per, m_per), :],
                w_ref[...],
                preferred_element_type=jnp.float32,
            )
            if s == 0:
                send_buf[...] = part
            else:
                recv = pltpu.make_async_remote_copy(
                    src_ref=send_buf,
                    dst_ref=recv_buf.at[s - 1],
                    send_sem=send_sems.at[s - 1],
                    recv_sem=recv_sems.at[s - 1],
                    device_id=(left,),
                    device_id_type=pl.DeviceIdType.MESH,
                )
                recv.wait_recv()
                send_buf[...] = recv_buf[s - 1] + part
            if s < N_DEV - 1:
                send = pltpu.make_async_remote_copy(
                    src_ref=send_buf,
                    dst_ref=recv_buf.at[s],
                    send_sem=send_sems.at[s],
                    recv_sem=recv_sems.at[s],
                    device_id=(right,),
                    device_id_type=pl.DeviceIdType.MESH,
                )
                send.start()
                send.wait_send()
            else:
                y = send_buf[...]
                out_ref[...] = y * jax.nn.sigmoid(y)

        @functools.partial(
            pl.run_scoped, second_barrier=pltpu.SemaphoreType.REGULAR
        )
        def _(second_barrier):
            for nbr in (left, right):
                pl.semaphore_signal(
                    second_barrier, inc=1,
                    device_id=(nbr,), device_id_type=pl.DeviceIdType.MESH,
                )
            pl.semaphore_wait(second_barrier, 2)

    return pl.pallas_call(
        body,
        out_shape=jax.ShapeDtypeStruct((m_per, n), jnp.float32),
        in_specs=[
            pl.BlockSpec(memory_space=pltpu.VMEM),
            pl.BlockSpec(memory_space=pltpu.VMEM),
        ],
        out_specs=pl.BlockSpec(memory_space=pltpu.VMEM),
        scratch_shapes=[
            pltpu.VMEM((m_per, n), jnp.float32),
            pltpu.VMEM((N_DEV - 1, m_per, n), jnp.float32),
            pltpu.SemaphoreType.DMA((N_DEV - 1,)),
            pltpu.SemaphoreType.DMA((N_DEV - 1,)),
        ],
        compiler_params=pltpu.CompilerParams(collective_id=0),
    )(x, w_mat)


# baseline (device time: 382720 ns/iter reference)
import functools

import jax
import jax.numpy as jnp
from jax import lax
from jax.experimental import pallas as pl
from jax.experimental.pallas import tpu as pltpu

N_DEV = 16


def kernel(x, w_mat):
    m_glob, k_per = x.shape
    _, n = w_mat.shape
    m_per = m_glob // N_DEV

    def body(x_ref, w_ref, out_ref, send_buf, recv_buf, send_sems, recv_sems):
        my = lax.axis_index("i")
        left = lax.rem(my + N_DEV - 1, N_DEV)
        right = lax.rem(my + 1, N_DEV)

        barrier_sem = pltpu.get_barrier_semaphore()
        for nbr in (left, right):
            pl.semaphore_signal(
                barrier_sem, inc=1,
                device_id=(nbr,), device_id_type=pl.DeviceIdType.MESH,
            )
        pl.semaphore_wait(barrier_sem, 2)

        for s in range(N_DEV):
            c = lax.rem(my + 2 * N_DEV - 1 - s, N_DEV)
            part = jnp.dot(
                x_ref[pl.ds(c * m_per, m_per), :],
                w_ref[...],
                preferred_element_type=jnp.float32,
            )
            if s == 0:
                send_buf[...] = part
            else:
                recv = pltpu.make_async_remote_copy(
                    src_ref=send_buf,
                    dst_ref=recv_buf.at[s - 1],
                    send_sem=send_sems.at[s - 1],
                    recv_sem=recv_sems.at[s - 1],
                    device_id=(left,),
                    device_id_type=pl.DeviceIdType.MESH,
                )
                recv.wait_recv()
                send_buf[...] = recv_buf[s - 1] + part
            if s < N_DEV - 1:
                send = pltpu.make_async_remote_copy(
                    src_ref=send_buf,
                    dst_ref=recv_buf.at[s],
                    send_sem=send_sems.at[s],
                    recv_sem=recv_sems.at[s],
                    device_id=(right,),
                    device_id_type=pl.DeviceIdType.MESH,
                )
                send.start()
                send.wait_send()
            else:
                y = send_buf[...]
                out_ref[...] = y * jax.nn.sigmoid(y)

        @functools.partial(
            pl.run_scoped, second_barrier=pltpu.SemaphoreType.REGULAR
        )
        def _(second_barrier):
            for nbr in (left, right):
                pl.semaphore_signal(
                    second_barrier, inc=1,
                    device_id=(nbr,), device_id_type=pl.DeviceIdType.MESH,
                )
            pl.semaphore_wait(second_barrier, 2)

    return pl.pallas_call(
        body,
        out_shape=jax.ShapeDtypeStruct((m_per, n), jnp.float32),
        in_specs=[
            pl.BlockSpec(memory_space=pltpu.VMEM),
            pl.BlockSpec(memory_space=pltpu.VMEM),
        ],
        out_specs=pl.BlockSpec(memory_space=pltpu.VMEM),
        scratch_shapes=[
            pltpu.VMEM((m_per, n), jnp.float32),
            pltpu.VMEM((N_DEV - 1, m_per, n), jnp.float32),
            pltpu.SemaphoreType.DMA((N_DEV - 1,)),
            pltpu.SemaphoreType.DMA((N_DEV - 1,)),
        ],
        compiler_params=pltpu.CompilerParams(
            collective_id=0,
            vmem_limit_bytes=100 * 1024 * 1024,
        ),
    )(x, w_mat)


# device time: 213931 ns/iter; 1.7890x vs baseline; 1.7890x over previous
import functools

import jax
import jax.numpy as jnp
from jax import lax
from jax.experimental import pallas as pl
from jax.experimental.pallas import tpu as pltpu

N_DEV = 16
SPLIT = 1


def kernel(x, w_mat):
    m_glob, k_per = x.shape
    _, n = w_mat.shape
    m_per = m_glob // N_DEV
    n_half = n // 2
    w_cols = n_half // SPLIT

    rings = [(+1, j * w_cols) for j in range(SPLIT)] + [
        (-1, n_half + j * w_cols) for j in range(SPLIT)
    ]
    R = len(rings)

    def body(x_ref, w_ref, out_ref, *scratch):
        bufs = scratch[0:R]
        ssems = scratch[R : 2 * R]
        rsems = scratch[2 * R : 3 * R]

        my = lax.axis_index("i")
        left = lax.rem(my + N_DEV - 1, N_DEV)
        right = lax.rem(my + 1, N_DEV)

        barrier_sem = pltpu.get_barrier_semaphore()
        for nbr in (left, right):
            pl.semaphore_signal(
                barrier_sem, inc=1,
                device_id=(nbr,), device_id_type=pl.DeviceIdType.MESH,
            )
        pl.semaphore_wait(barrier_sem, 2)

        sends = []
        for s in range(N_DEV):
            slot = N_DEV - 1 if s == 0 else s - 1
            for dirn in (+1, -1):
                if dirn == +1:
                    c = lax.rem(my + 2 * N_DEV - 1 - s, N_DEV)
                    src_nbr, dst_nbr = left, right
                    dir_col0 = 0
                else:
                    c = lax.rem(my + 1 + s, N_DEV)
                    src_nbr, dst_nbr = right, left
                    dir_col0 = n_half
                part = jnp.dot(
                    x_ref[pl.ds(c * m_per, m_per), :],
                    w_ref[:, dir_col0 : dir_col0 + n_half],
                    preferred_element_type=jnp.float32,
                )
                for r, (rd, col0) in enumerate(rings):
                    if rd != dirn:
                        continue
                    sub = part[:, col0 - dir_col0 : col0 - dir_col0 + w_cols]
                    if s == 0:
                        bufs[r][slot] = sub
                    else:
                        recv = pltpu.make_async_remote_copy(
                            src_ref=bufs[r].at[N_DEV - 1],
                            dst_ref=bufs[r].at[s - 1],
                            send_sem=ssems[r].at[s - 1],
                            recv_sem=rsems[r].at[s - 1],
                            device_id=(src_nbr,),
                            device_id_type=pl.DeviceIdType.MESH,
                        )
                        recv.wait_recv()
                        bufs[r][s - 1] = bufs[r][s - 1] + sub
                    if s < N_DEV - 1:
                        send = pltpu.make_async_remote_copy(
                            src_ref=bufs[r].at[slot],
                            dst_ref=bufs[r].at[s],
                            send_sem=ssems[r].at[s],
                            recv_sem=rsems[r].at[s],
                            device_id=(dst_nbr,),
                            device_id_type=pl.DeviceIdType.MESH,
                        )
                        send.start()
                        sends.append(send)
                    else:
                        y = bufs[r][slot]
                        out_ref[:, col0 : col0 + w_cols] = (
                            y * jax.nn.sigmoid(y)
                        )

        for send in sends:
            send.wait_send()

        @functools.partial(
            pl.run_scoped, second_barrier=pltpu.SemaphoreType.REGULAR
        )
        def _(second_barrier):
            for nbr in (left, right):
                pl.semaphore_signal(
                    second_barrier, inc=1,
                    device_id=(nbr,), device_id_type=pl.DeviceIdType.MESH,
                )
            pl.semaphore_wait(second_barrier, 2)

    return pl.pallas_call(
        body,
        out_shape=jax.ShapeDtypeStruct((m_per, n), jnp.float32),
        in_specs=[
            pl.BlockSpec(memory_space=pltpu.VMEM),
            pl.BlockSpec(memory_space=pltpu.VMEM),
        ],
        out_specs=pl.BlockSpec(memory_space=pltpu.VMEM),
        scratch_shapes=(
            [pltpu.VMEM((N_DEV, m_per, w_cols), jnp.float32) for _ in range(R)]
            + [pltpu.SemaphoreType.DMA((N_DEV - 1,)) for _ in range(R)]
            + [pltpu.SemaphoreType.DMA((N_DEV - 1,)) for _ in range(R)]
        ),
        compiler_params=pltpu.CompilerParams(
            collective_id=0,
            vmem_limit_bytes=100 * 1024 * 1024,
        ),
    )(x, w_mat)


# device time: 190749 ns/iter; 2.0064x vs baseline; 1.1215x over previous
import functools

import jax
import jax.numpy as jnp
from jax import lax
from jax.experimental import pallas as pl
from jax.experimental.pallas import tpu as pltpu

N_DEV = 16
SPLIT = 2


def kernel(x, w_mat):
    m_glob, k_per = x.shape
    _, n = w_mat.shape
    m_per = m_glob // N_DEV
    n_half = n // 2
    w_cols = n_half // SPLIT

    rings = [(+1, j * w_cols) for j in range(SPLIT)] + [
        (-1, n_half + j * w_cols) for j in range(SPLIT)
    ]
    R = len(rings)

    def body(x_ref, w_ref, out_ref, *scratch):
        bufs = scratch[0:R]
        ssems = scratch[R : 2 * R]
        rsems = scratch[2 * R : 3 * R]

        my = lax.axis_index("i")
        left = lax.rem(my + N_DEV - 1, N_DEV)
        right = lax.rem(my + 1, N_DEV)

        barrier_sem = pltpu.get_barrier_semaphore()
        for nbr in (left, right):
            pl.semaphore_signal(
                barrier_sem, inc=1,
                device_id=(nbr,), device_id_type=pl.DeviceIdType.MESH,
            )
        pl.semaphore_wait(barrier_sem, 2)

        sends = []
        for s in range(N_DEV):
            slot = N_DEV - 1 if s == 0 else s - 1
            for dirn in (+1, -1):
                if dirn == +1:
                    c = lax.rem(my + 2 * N_DEV - 1 - s, N_DEV)
                    src_nbr, dst_nbr = left, right
                    dir_col0 = 0
                else:
                    c = lax.rem(my + 1 + s, N_DEV)
                    src_nbr, dst_nbr = right, left
                    dir_col0 = n_half
                part = jnp.dot(
                    x_ref[pl.ds(c * m_per, m_per), :],
                    w_ref[:, dir_col0 : dir_col0 + n_half],
                    preferred_element_type=jnp.float32,
                )
                for r, (rd, col0) in enumerate(rings):
                    if rd != dirn:
                        continue
                    sub = part[:, col0 - dir_col0 : col0 - dir_col0 + w_cols]
                    if s == 0:
                        bufs[r][slot] = sub
                    else:
                        recv = pltpu.make_async_remote_copy(
                            src_ref=bufs[r].at[N_DEV - 1],
                            dst_ref=bufs[r].at[s - 1],
                            send_sem=ssems[r].at[s - 1],
                            recv_sem=rsems[r].at[s - 1],
                            device_id=(src_nbr,),
                            device_id_type=pl.DeviceIdType.MESH,
                        )
                        recv.wait_recv()
                        bufs[r][s - 1] = bufs[r][s - 1] + sub
                    if s < N_DEV - 1:
                        send = pltpu.make_async_remote_copy(
                            src_ref=bufs[r].at[slot],
                            dst_ref=bufs[r].at[s],
                            send_sem=ssems[r].at[s],
                            recv_sem=rsems[r].at[s],
                            device_id=(dst_nbr,),
                            device_id_type=pl.DeviceIdType.MESH,
                        )
                        send.start()
                        sends.append(send)
                    else:
                        y = bufs[r][slot]
                        out_ref[:, col0 : col0 + w_cols] = (
                            y * jax.nn.sigmoid(y)
                        )

        for send in sends:
            send.wait_send()

        @functools.partial(
            pl.run_scoped, second_barrier=pltpu.SemaphoreType.REGULAR
        )
        def _(second_barrier):
            for nbr in (left, right):
                pl.semaphore_signal(
                    second_barrier, inc=1,
                    device_id=(nbr,), device_id_type=pl.DeviceIdType.MESH,
                )
            pl.semaphore_wait(second_barrier, 2)

    return pl.pallas_call(
        body,
        out_shape=jax.ShapeDtypeStruct((m_per, n), jnp.float32),
        in_specs=[
            pl.BlockSpec(memory_space=pltpu.VMEM),
            pl.BlockSpec(memory_space=pltpu.VMEM),
        ],
        out_specs=pl.BlockSpec(memory_space=pltpu.VMEM),
        scratch_shapes=(
            [pltpu.VMEM((N_DEV, m_per, w_cols), jnp.float32) for _ in range(R)]
            + [pltpu.SemaphoreType.DMA((N_DEV - 1,)) for _ in range(R)]
            + [pltpu.SemaphoreType.DMA((N_DEV - 1,)) for _ in range(R)]
        ),
        compiler_params=pltpu.CompilerParams(
            collective_id=0,
            vmem_limit_bytes=100 * 1024 * 1024,
        ),
    )(x, w_mat)
